# baseline (speedup 1.0000x reference)
import functools

import jax
import jax.numpy as jnp
from jax import lax
from jax.experimental import pallas as pl
from jax.experimental.pallas import tpu as pltpu
from jax.experimental.pallas import tpu_sc as plsc

BATCH = 16384
EMBED_DIM = 64
PACKED = 2 * EMBED_DIM
LANES = 16
NUM_CORES = 2
NUM_SUBCORES = 16
NUM_WORKERS = NUM_CORES * NUM_SUBCORES
BPW = BATCH // NUM_WORKERS
CHUNK = 128
NCHUNK = BPW // CHUNK
STAGE_ROWS = 256
NSTAGE = BPW // STAGE_ROWS


def _mf_body(upk_ref, ipk_ref, ui_ref, ii_ref, ue_ref, ie_ref, bsum_ref, out_ref,
             uidx_v, iidx_v, urows_v, irows_v, bias_v, out_v,
             uraw_v, iraw_v, sem):
    wid = lax.axis_index("s") * NUM_CORES + lax.axis_index("c")
    base = wid * BPW

    pltpu.sync_copy(upk_ref.at[pl.ds(wid * NCHUNK, NCHUNK)], uidx_v)
    pltpu.sync_copy(ipk_ref.at[pl.ds(wid * NCHUNK, NCHUNK)], iidx_v)
    pltpu.sync_copy(ui_ref.at[pl.ds(base, BPW)], uraw_v)
    pltpu.sync_copy(ii_ref.at[pl.ds(base, BPW)], iraw_v)
    pltpu.sync_copy(bsum_ref.at[pl.ds(base, BPW)], bias_v)

    lane_iota = lax.iota(jnp.int32, LANES)
    perms = [(lane_iota + s) & (LANES - 1) for s in (8, 4, 2, 1)]
    onehots = [
        jnp.where(lane_iota == k, jnp.float32(1.0), jnp.float32(0.0))
        for k in range(LANES)
    ]

    for stage in range(NSTAGE):
        copies = []
        for j in range(NCHUNK // NSTAGE):
            c = stage * (NCHUNK // NSTAGE) + j
            sl = pl.ds(j * CHUNK, CHUNK)
            copies.append(
                pltpu.async_copy(ue_ref.at[uidx_v.at[c]], urows_v.at[sl], sem))
            copies.append(
                pltpu.async_copy(ie_ref.at[iidx_v.at[c]], irows_v.at[sl], sem))
        for cp in copies:
            cp.wait()

        s0 = stage * STAGE_ROWS
        zero_vec = lane_iota & 0

        def group(g, carry):
            r0 = g * LANES
            acc = bias_v[pl.ds(s0 + r0, LANES)]
            hu_f = (uraw_v[pl.ds(s0 + r0, LANES)] & 1).astype(jnp.float32)
            hi_f = (iraw_v[pl.ds(s0 + r0, LANES)] & 1).astype(jnp.float32)
            for l in range(LANES):
                r = r0 + l
                bcast = zero_vec + l
                mu = hu_f.at[bcast].get(mode="promise_in_bounds")
                mi = hi_f.at[bcast].get(mode="promise_in_bounds")
                p = None
                for k in range(EMBED_DIM // LANES):
                    ulo = urows_v[r, pl.ds(k * LANES, LANES)]
                    uhi = urows_v[r, pl.ds(EMBED_DIM + k * LANES, LANES)]
                    vlo = irows_v[r, pl.ds(k * LANES, LANES)]
                    vhi = irows_v[r, pl.ds(EMBED_DIM + k * LANES, LANES)]
                    u = ulo + (uhi - ulo) * mu
                    v = vlo + (vhi - vlo) * mi
                    p = u * v if p is None else p + u * v
                for perm in perms:
                    p = p + p.at[perm].get(mode="promise_in_bounds",
                                           unique_indices=True)
                acc = acc + p * onehots[l]
            out_v[pl.ds(s0 + r0, LANES)] = acc
            return carry

        lax.fori_loop(0, STAGE_ROWS // LANES, group, 0)

    pltpu.sync_copy(out_v, out_ref.at[pl.ds(base, BPW)])


@functools.partial(
    pl.kernel,
    out_type=jax.ShapeDtypeStruct((BATCH,), jnp.float32),
    mesh=plsc.VectorSubcoreMesh(core_axis_name="c", subcore_axis_name="s"),
    scratch_types=[
        pltpu.VMEM((NCHUNK, CHUNK), jnp.int32),
        pltpu.VMEM((NCHUNK, CHUNK), jnp.int32),
        pltpu.VMEM((STAGE_ROWS, PACKED), jnp.float32),
        pltpu.VMEM((STAGE_ROWS, PACKED), jnp.float32),
        pltpu.VMEM((BPW,), jnp.float32),
        pltpu.VMEM((BPW,), jnp.float32),
        pltpu.VMEM((BPW,), jnp.int32),
        pltpu.VMEM((BPW,), jnp.int32),
        pltpu.SemaphoreType.DMA,
    ],
)
def _mf_kernel(upk, ipk, ui, ii, ue, ie, bsum, out,
               uidx_v, iidx_v, urows_v, irows_v, bias_v, out_v,
               uraw_v, iraw_v, sem):
    _mf_body(upk, ipk, ui, ii, ue, ie, bsum, out,
             uidx_v, iidx_v, urows_v, irows_v, bias_v, out_v,
             uraw_v, iraw_v, sem)


def kernel(user_indices, item_indices, user_emb, item_emb, user_bias, item_bias):
    ui = user_indices.astype(jnp.int32)
    ii = item_indices.astype(jnp.int32)
    upk = (ui >> 1).reshape(NUM_WORKERS * NCHUNK, CHUNK)
    ipk = (ii >> 1).reshape(NUM_WORKERS * NCHUNK, CHUNK)
    ue = user_emb.reshape(NUM_USERS_PACKED, PACKED)
    ie = item_emb.reshape(NUM_USERS_PACKED, PACKED)
    user_b = jnp.take(user_bias, user_indices, axis=0).squeeze(-1)
    item_b = jnp.take(item_bias, item_indices, axis=0).squeeze(-1)
    bsum = user_b + item_b
    return _mf_kernel(upk, ipk, ui, ii, ue, ie, bsum)


NUM_USERS_PACKED = 500000


# speedup vs baseline: 1.0723x; 1.0723x over previous
import functools

import jax
import jax.numpy as jnp
from jax import lax
from jax.experimental import pallas as pl
from jax.experimental.pallas import tpu as pltpu
from jax.experimental.pallas import tpu_sc as plsc

BATCH = 16384
EMBED_DIM = 64
PACKED = 128
LANES = 16
NUM_CORES = 2
NUM_SUBCORES = 16
NUM_WORKERS = NUM_CORES * NUM_SUBCORES
BPW = BATCH // NUM_WORKERS
CHUNK = 128
NCHUNK = BPW // CHUNK
STAGE_ROWS = 256
NSTAGE = BPW // STAGE_ROWS


def _mf_body(ui_ref, ii_ref, ue_ref, ie_ref, bsum_ref, out_ref,
             uidx_v, iidx_v, urows_v, irows_v, bias_v, out_v, sem):
    wid = lax.axis_index("s") * NUM_CORES + lax.axis_index("c")
    base = wid * BPW

    pltpu.sync_copy(ui_ref.at[pl.ds(wid * NCHUNK, NCHUNK)], uidx_v)
    pltpu.sync_copy(ii_ref.at[pl.ds(wid * NCHUNK, NCHUNK)], iidx_v)
    pltpu.sync_copy(bsum_ref.at[pl.ds(base, BPW)], bias_v)

    lane_iota = lax.iota(jnp.int32, LANES)
    perms = [(lane_iota + s) & (LANES - 1) for s in (8, 4, 2, 1)]
    onehots = [
        jnp.where(lane_iota == k, jnp.float32(1.0), jnp.float32(0.0))
        for k in range(LANES)
    ]

    for stage in range(NSTAGE):
        copies = []
        for j in range(NCHUNK // NSTAGE):
            c = stage * (NCHUNK // NSTAGE) + j
            sl = pl.ds(j * CHUNK, CHUNK)
            copies.append(
                pltpu.async_copy(ue_ref.at[uidx_v.at[c]], urows_v.at[sl], sem))
            copies.append(
                pltpu.async_copy(ie_ref.at[iidx_v.at[c]], irows_v.at[sl], sem))
        for cp in copies:
            cp.wait()

        s0 = stage * STAGE_ROWS

        def group(g, carry):
            r0 = g * LANES
            acc = bias_v[pl.ds(s0 + r0, LANES)]
            for l in range(LANES):
                r = r0 + l
                p = None
                for k in range(EMBED_DIM // LANES):
                    u = urows_v[r, pl.ds(k * LANES, LANES)]
                    v = irows_v[r, pl.ds(k * LANES, LANES)]
                    p = u * v if p is None else p + u * v
                for perm in perms:
                    p = p + p.at[perm].get(mode="promise_in_bounds",
                                           unique_indices=True)
                acc = acc + p * onehots[l]
            out_v[pl.ds(s0 + r0, LANES)] = acc
            return carry

        lax.fori_loop(0, STAGE_ROWS // LANES, group, 0)

    pltpu.sync_copy(out_v, out_ref.at[pl.ds(base, BPW)])


@functools.partial(
    pl.kernel,
    out_type=jax.ShapeDtypeStruct((BATCH,), jnp.float32),
    mesh=plsc.VectorSubcoreMesh(core_axis_name="c", subcore_axis_name="s"),
    scratch_types=[
        pltpu.VMEM((NCHUNK, CHUNK), jnp.int32),
        pltpu.VMEM((NCHUNK, CHUNK), jnp.int32),
        pltpu.VMEM((STAGE_ROWS, PACKED), jnp.float32),
        pltpu.VMEM((STAGE_ROWS, PACKED), jnp.float32),
        pltpu.VMEM((BPW,), jnp.float32),
        pltpu.VMEM((BPW,), jnp.float32),
        pltpu.SemaphoreType.DMA,
    ],
)
def _mf_kernel(ui, ii, ue, ie, bsum, out,
               uidx_v, iidx_v, urows_v, irows_v, bias_v, out_v, sem):
    _mf_body(ui, ii, ue, ie, bsum, out,
             uidx_v, iidx_v, urows_v, irows_v, bias_v, out_v, sem)


def kernel(user_indices, item_indices, user_emb, item_emb, user_bias, item_bias):
    ui = user_indices.astype(jnp.int32).reshape(NUM_WORKERS * NCHUNK, CHUNK)
    ii = item_indices.astype(jnp.int32).reshape(NUM_WORKERS * NCHUNK, CHUNK)
    ue = jnp.pad(user_emb, ((0, 0), (0, PACKED - EMBED_DIM)))
    ie = jnp.pad(item_emb, ((0, 0), (0, PACKED - EMBED_DIM)))
    user_b = jnp.take(user_bias, user_indices, axis=0).squeeze(-1)
    item_b = jnp.take(item_bias, item_indices, axis=0).squeeze(-1)
    bsum = user_b + item_b
    return _mf_kernel(ui, ii, ue, ie, bsum)
